# baseline (speedup 1.0000x reference)
import jax
import jax.numpy as jnp
from jax import lax
from jax.experimental import pallas as pl
from jax.experimental.pallas import tpu as pltpu
from jax.experimental.pallas import tpu_sc as plsc

N = 10000
E = 320000
D = 128
ED = 16

NC = 2
NS = 16
NW = NC * NS
CH = 128
RPS = 624
RTAIL = N - NS * RPS
RCH = 48


def _gelu(x):
    return 0.5 * x * (1.0 + lax.erf(x * jnp.float32(0.7071067811865476)))


CHG = 48


def _sc_gather(src, dst, h, e0, ne):
    epw = ne // NW
    nfull = epw // CHG
    tail = epw - nfull * CHG

    def body(src_hbm, dst_hbm, h_hbm, hs_hbm, hd_hbm,
             six, dix, bs0, bs1, bs2, bd0, bd1, bd2,
             idx_st, idx_dt, h_sh,
             gsem0, gsem1, gsem2, wsem0, wsem1, wsem2, ssem):
        s = lax.axis_index("s")
        wid = s * NC + lax.axis_index("c")
        base = e0 + wid * epw
        obase = wid * epw
        bs = (bs0, bs1, bs2)
        bd = (bd0, bd1, bd2)
        gsem = (gsem0, gsem1, gsem2)
        wsem = (wsem0, wsem1, wsem2)

        pend_s = [None, None]
        for k in range(RPS // RCH):
            p = k % 2
            if pend_s[p] is not None:
                pend_s[p][1].wait()
            roff = s * RPS + k * RCH
            rd = pltpu.async_copy(h_hbm.at[pl.ds(roff, RCH)],
                                  bs[p].at[pl.ds(0, RCH)], gsem[p])
            rd.wait()
            pend_s[p] = (roff, pltpu.async_copy(
                bs[p].at[pl.ds(0, RCH)], h_sh.at[pl.ds(roff, RCH)], ssem))
        for p in range(2):
            if pend_s[p] is not None:
                pend_s[p][1].wait()

        @pl.when(s == 0)
        def _():
            roff = NS * RPS
            pltpu.sync_copy(h_hbm.at[pl.ds(roff, RTAIL)],
                            bs0.at[pl.ds(0, RTAIL)])
            pltpu.sync_copy(bs0.at[pl.ds(0, RTAIL)],
                            h_sh.at[pl.ds(roff, RTAIL)])

        pltpu.sync_copy(src_hbm.at[pl.ds(base, epw)], six)
        pltpu.sync_copy(dst_hbm.at[pl.ds(base, epw)], dix)
        plsc.subcore_barrier()

        pend_g = [None, None, None]
        pend_w = [None, None, None]

        def fire_gather(j, p):
            isl_s = six.at[pl.ds(j * CHG, CHG)]
            isl_d = dix.at[pl.ds(j * CHG, CHG)]
            pend_g[p] = (pltpu.async_copy(h_sh.at[isl_s], bs[p], gsem[p]),
                         pltpu.async_copy(h_sh.at[isl_d], bd[p], gsem[p]))

        def fire_writes(j, p):
            off = obase + j * CHG
            pend_w[p] = (
                pltpu.async_copy(bs[p], hs_hbm.at[pl.ds(off, CHG)], wsem[p]),
                pltpu.async_copy(bd[p], hd_hbm.at[pl.ds(off, CHG)], wsem[p]))

        fire_gather(0, 0)
        if nfull > 1:
            fire_gather(1, 1)
        for j in range(nfull):
            p = j % 3
            for dsc in pend_g[p]:
                dsc.wait()
            fire_writes(j, p)
            nj = j + 2
            if nj < nfull:
                q = nj % 3
                if pend_w[q] is not None:
                    for dsc in pend_w[q]:
                        dsc.wait()
                    pend_w[q] = None
                fire_gather(nj, q)
        for p in range(3):
            if pend_w[p] is not None:
                for dsc in pend_w[p]:
                    dsc.wait()

        if tail:
            off = nfull * CHG
            pltpu.sync_copy(src_hbm.at[pl.ds(base + off, tail)], idx_st)
            pltpu.sync_copy(dst_hbm.at[pl.ds(base + off, tail)], idx_dt)
            cps = pltpu.async_copy(h_sh.at[idx_st], bs0.at[pl.ds(0, tail)],
                                   gsem0)
            cpd = pltpu.async_copy(h_sh.at[idx_dt], bd0.at[pl.ds(0, tail)],
                                   gsem1)
            cps.wait()
            cpd.wait()
            pltpu.sync_copy(bs0.at[pl.ds(0, tail)],
                            hs_hbm.at[pl.ds(obase + off, tail)])
            pltpu.sync_copy(bd0.at[pl.ds(0, tail)],
                            hd_hbm.at[pl.ds(obase + off, tail)])

    mesh = plsc.VectorSubcoreMesh(core_axis_name="c", subcore_axis_name="s")
    f = pl.kernel(
        body,
        out_type=[jax.ShapeDtypeStruct((ne, D), jnp.float32),
                  jax.ShapeDtypeStruct((ne, D), jnp.float32)],
        mesh=mesh,
        scratch_types=[
            pltpu.VMEM((epw,), jnp.int32),
            pltpu.VMEM((epw,), jnp.int32),
            pltpu.VMEM((CHG, D), jnp.float32),
            pltpu.VMEM((CHG, D), jnp.float32),
            pltpu.VMEM((CHG, D), jnp.float32),
            pltpu.VMEM((CHG, D), jnp.float32),
            pltpu.VMEM((CHG, D), jnp.float32),
            pltpu.VMEM((CHG, D), jnp.float32),
            pltpu.VMEM((max(tail, 8),), jnp.int32),
            pltpu.VMEM((max(tail, 8),), jnp.int32),
            pltpu.VMEM_SHARED((N, D), jnp.float32),
            pltpu.SemaphoreType.DMA,
            pltpu.SemaphoreType.DMA,
            pltpu.SemaphoreType.DMA,
            pltpu.SemaphoreType.DMA,
            pltpu.SemaphoreType.DMA,
            pltpu.SemaphoreType.DMA,
            pltpu.SemaphoreType.DMA,
        ],
    )
    return f(src, dst, h)


def _sc_scatter(dst, m, e0, ne):
    epw = ne // NW
    nfull = epw // CH
    tail = epw - nfull * CH

    def body(dst_hbm, m_hbm, zrow_hbm, zcnt_hbm, one_hbm, onet_hbm,
             aggp_hbm, cntp_hbm,
             idx0, idx1, mb0, mb1, ones_v, idx_t, ones_t,
             st0, st1, cstage, agg_sh, cnt_sh,
             msem0, msem1, asem0, asem1, osem0, osem1):
        c = lax.axis_index("c")
        s = lax.axis_index("s")
        base = e0 + c * (ne // NC) + s * epw
        idxb = (idx0, idx1)
        mb = (mb0, mb1)
        st = (st0, st1)
        msem = (msem0, msem1)
        asem = (asem0, asem1)
        osem = (osem0, osem1)

        pltpu.sync_copy(zrow_hbm, st0)
        pltpu.sync_copy(zcnt_hbm, cstage)
        zp = [pltpu.async_copy(cstage, cnt_sh.at[pl.ds(s * RPS, RPS)], osem1)]
        for k in range(RPS // RCH):
            zp.append(pltpu.async_copy(
                st0, agg_sh.at[pl.ds(s * RPS + k * RCH, RCH)], osem0))

        @pl.when(s == 0)
        def _():
            pltpu.sync_copy(st0.at[pl.ds(0, RTAIL)],
                            agg_sh.at[pl.ds(NS * RPS, RTAIL)])
            pltpu.sync_copy(cstage.at[pl.ds(0, RTAIL)],
                            cnt_sh.at[pl.ds(NS * RPS, RTAIL)])

        pltpu.sync_copy(one_hbm, ones_v)
        pltpu.sync_copy(onet_hbm, ones_t)
        for dsc in zp:
            dsc.wait()
        plsc.subcore_barrier()

        pend_in = [None, None]
        pend_add = [None, None]

        def fire_in(j, p):
            off = base + j * CH
            pend_in[p] = (
                pltpu.async_copy(dst_hbm.at[pl.ds(off, CH)], idxb[p], msem[p]),
                pltpu.async_copy(m_hbm.at[pl.ds(j * CH + base - e0, CH)],
                                 mb[p], msem[p]))

        def fire_add(j, p):
            pend_add[p] = (
                pltpu.async_copy(mb[p], agg_sh.at[idxb[p]], asem[p], add=True),
                pltpu.async_copy(ones_v, cnt_sh.at[idxb[p]], asem[p],
                                 add=True))

        fire_in(0, 0)
        if nfull > 1:
            fire_in(1, 1)
        for j in range(nfull):
            p = j % 2
            for dsc in pend_in[p]:
                dsc.wait()
            if pend_add[p] is not None:
                for dsc in pend_add[p]:
                    dsc.wait()
            fire_add(j, p)
            nj = j + 2
            if nj < nfull:
                for dsc in pend_add[p]:
                    dsc.wait()
                pend_add[p] = None
                fire_in(nj, p)
        for p in range(2):
            if pend_add[p] is not None:
                for dsc in pend_add[p]:
                    dsc.wait()

        if tail:
            off = base + nfull * CH
            pltpu.sync_copy(dst_hbm.at[pl.ds(off, tail)], idx_t)
            pltpu.sync_copy(m_hbm.at[pl.ds(off - e0, tail)],
                            mb0.at[pl.ds(0, tail)])
            pltpu.sync_copy(mb0.at[pl.ds(0, tail)], agg_sh.at[idx_t],
                            add=True)
            pltpu.sync_copy(ones_t, cnt_sh.at[idx_t], add=True)

        plsc.subcore_barrier()

        pend_rd = [None, None]

        def fire_rd(k, p):
            pend_rd[p] = pltpu.async_copy(
                agg_sh.at[pl.ds(s * RPS + k * RCH, RCH)], st[p], msem[p])

        pend_wr = [None, None]
        fire_rd(0, 0)
        fire_rd(1, 1)
        for k in range(RPS // RCH):
            p = k % 2
            pend_rd[p].wait()
            if pend_wr[p] is not None:
                pend_wr[p].wait()
            pend_wr[p] = pltpu.async_copy(
                st[p], aggp_hbm.at[c, pl.ds(s * RPS + k * RCH, RCH)], osem[p])
            nk = k + 2
            if nk < RPS // RCH:
                pend_wr[p].wait()
                pend_wr[p] = None
                fire_rd(nk, p)
        for p in range(2):
            if pend_wr[p] is not None:
                pend_wr[p].wait()
        pltpu.sync_copy(cnt_sh.at[pl.ds(s * RPS, RPS)], cstage)
        pltpu.sync_copy(cstage, cntp_hbm.at[pl.ds(c * N + s * RPS, RPS)])

        @pl.when(s == 0)
        def _():
            pltpu.sync_copy(agg_sh.at[pl.ds(NS * RPS, RTAIL)],
                            st0.at[pl.ds(0, RTAIL)])
            pltpu.sync_copy(st0.at[pl.ds(0, RTAIL)],
                            aggp_hbm.at[c, pl.ds(NS * RPS, RTAIL)])
            pltpu.sync_copy(cnt_sh.at[pl.ds(NS * RPS, RTAIL)],
                            cstage.at[pl.ds(0, RTAIL)])
            pltpu.sync_copy(cstage.at[pl.ds(0, RTAIL)],
                            cntp_hbm.at[pl.ds(c * N + NS * RPS, RTAIL)])

    zrow = jnp.zeros((RCH, D), jnp.float32)
    zcnt = jnp.zeros((RPS,), jnp.float32)
    one = jnp.ones((CH,), jnp.float32)
    onet = jnp.ones((max(tail, 8),), jnp.float32)
    mesh = plsc.VectorSubcoreMesh(core_axis_name="c", subcore_axis_name="s")
    f = pl.kernel(
        body,
        out_type=[jax.ShapeDtypeStruct((NC, N, D), jnp.float32),
                  jax.ShapeDtypeStruct((NC * N,), jnp.float32)],
        mesh=mesh,
        scratch_types=[
            pltpu.VMEM((CH,), jnp.int32),
            pltpu.VMEM((CH,), jnp.int32),
            pltpu.VMEM((CH, D), jnp.float32),
            pltpu.VMEM((CH, D), jnp.float32),
            pltpu.VMEM((CH,), jnp.float32),
            pltpu.VMEM((max(tail, 8),), jnp.int32),
            pltpu.VMEM((max(tail, 8),), jnp.float32),
            pltpu.VMEM((RCH, D), jnp.float32),
            pltpu.VMEM((RCH, D), jnp.float32),
            pltpu.VMEM((RPS,), jnp.float32),
            pltpu.VMEM_SHARED((N, D), jnp.float32),
            pltpu.VMEM_SHARED((N,), jnp.float32),
            pltpu.SemaphoreType.DMA,
            pltpu.SemaphoreType.DMA,
            pltpu.SemaphoreType.DMA,
            pltpu.SemaphoreType.DMA,
            pltpu.SemaphoreType.DMA,
            pltpu.SemaphoreType.DMA,
        ],
    )
    return f(dst, m, zrow, zcnt, one, onet)


def _edge_mlp_body(hs, hd, ea, w1s, w1d, w1e, b1, w2, b2, w3, b3, out):
    bf = jnp.bfloat16
    x = (jnp.dot(hs[...].astype(bf), w1s[...].astype(bf),
                 preferred_element_type=jnp.float32)
         + jnp.dot(hd[...].astype(bf), w1d[...].astype(bf),
                   preferred_element_type=jnp.float32)
         + jnp.dot(ea[...].astype(bf), w1e[...].astype(bf),
                   preferred_element_type=jnp.float32)
         + b1[...])
    x = _gelu(x)
    x = _gelu(jnp.dot(x.astype(bf), w2[...].astype(bf),
                      preferred_element_type=jnp.float32) + b2[...])
    out[...] = jnp.dot(x.astype(bf), w3[...].astype(bf),
                       preferred_element_type=jnp.float32) + b3[...]


def _tc_edge_mlp(hs, hd, ea, W1, b1, W2, b2, W3, b3, BE, ne, e0):
    grid = (ne // BE,)
    eoff = e0 // BE
    w1s, w1d, w1e = W1[:D], W1[D:2 * D], W1[2 * D:]
    full = lambda shape: pl.BlockSpec(shape, lambda i: (0, 0))
    return pl.pallas_call(
        _edge_mlp_body,
        grid=grid,
        in_specs=[
            pl.BlockSpec((BE, D), lambda i: (i, 0)),
            pl.BlockSpec((BE, D), lambda i: (i, 0)),
            pl.BlockSpec((BE, ED), lambda i: (i + eoff, 0)),
            full((D, D)), full((D, D)), full((ED, D)), full((1, D)),
            full((D, D)), full((1, D)),
            full((D, D)), full((1, D)),
        ],
        out_specs=pl.BlockSpec((BE, D), lambda i: (i, 0)),
        out_shape=jax.ShapeDtypeStruct((ne, D), jnp.float32),
    )(hs, hd, ea, w1s, w1d, w1e, b1.reshape(1, D), W2, b2.reshape(1, D),
      W3, b3.reshape(1, D))


def _node_body(h, a0, a1, a2, a3, c0, c1, c2, c3,
               u1h, u1a, ub1, u2, ub2, lw, lb, out):
    cnt = c0[...] + c1[...] + c2[...] + c3[...] + jnp.float32(1e-8)
    agg = (a0[...] + a1[...] + a2[...] + a3[...]) / cnt
    u = _gelu(jnp.dot(h[...], u1h[...], preferred_element_type=jnp.float32)
              + jnp.dot(agg, u1a[...], preferred_element_type=jnp.float32)
              + ub1[...])
    x = jnp.dot(u, u2[...], preferred_element_type=jnp.float32) + ub2[...] + h[...]
    mu = jnp.mean(x, axis=-1, keepdims=True)
    xc = x - mu
    var = jnp.mean(xc * xc, axis=-1, keepdims=True)
    out[...] = xc * lax.rsqrt(var + jnp.float32(1e-5)) * lw[...] + lb[...]


def _tc_node(h, aggs, cnts, U1, ub1, U2, ub2, ln_w, ln_b):
    BN = 2000
    grid = (N // BN,)
    u1h, u1a = U1[:D], U1[D:]
    full = lambda shape: pl.BlockSpec(shape, lambda i: (0, 0))
    row = pl.BlockSpec((BN, D), lambda i: (i, 0))
    col = pl.BlockSpec((BN, 1), lambda i: (i, 0))
    return pl.pallas_call(
        _node_body,
        grid=grid,
        in_specs=[row] + [row] * 4 + [col] * 4 + [
                  full((D, D)), full((D, D)), full((1, D)),
                  full((D, D)), full((1, D)),
                  full((1, D)), full((1, D))],
        out_specs=row,
        out_shape=jax.ShapeDtypeStruct((N, D), jnp.float32),
    )(h, *aggs, *cnts,
      u1h, u1a, ub1.reshape(1, D), U2, ub2.reshape(1, D),
      ln_w.reshape(1, D), ln_b.reshape(1, D))


def kernel(h, edge_index, edge_attr, W1, b1, W2, b2, W3, b3, U1, ub1, U2, ub2,
           ln_w, ln_b):
    src = edge_index[0]
    dst = edge_index[1]
    chunks = ((0, 160000, 2000), (160000, 160000, 2000))
    aggs, cnts = [], []
    for e0, ne, be in chunks:
        hs, hd = _sc_gather(src, dst, h, e0, ne)
        m = _tc_edge_mlp(hs, hd, edge_attr, W1, b1, W2, b2, W3,
                         b3, be, ne, e0)
        aggp, cntp = _sc_scatter(dst, m, e0, ne)
        aggs += [aggp[0], aggp[1]]
        cnts += [cntp[:N].reshape(N, 1), cntp[N:].reshape(N, 1)]
    return _tc_node(h, aggs, cnts, U1, ub1, U2, ub2, ln_w, ln_b)


# speedup vs baseline: 1.2245x; 1.2245x over previous
import jax
import jax.numpy as jnp
from jax import lax
from jax.experimental import pallas as pl
from jax.experimental.pallas import tpu as pltpu
from jax.experimental.pallas import tpu_sc as plsc

N = 10000
E = 320000
D = 128
ED = 16

NC = 2
NS = 16
NW = NC * NS
CH = 128
RPS = 624
RTAIL = N - NS * RPS
RCH = 48


def _gelu(x):
    return 0.5 * x * (1.0 + lax.erf(x * jnp.float32(0.7071067811865476)))


CHG = 48


def _sc_gather(src, dst, h, e0, ne):
    epw = ne // NW
    nfull = epw // CHG
    tail = epw - nfull * CHG

    def body(src_hbm, dst_hbm, h_hbm, hs_hbm, hd_hbm,
             six, dix, bs0, bs1, bs2, bd0, bd1, bd2,
             idx_st, idx_dt, h_sh,
             gsem0, gsem1, gsem2, wsem0, wsem1, wsem2, ssem):
        s = lax.axis_index("s")
        wid = s * NC + lax.axis_index("c")
        base = e0 + wid * epw
        obase = wid * epw
        bs = (bs0, bs1, bs2)
        bd = (bd0, bd1, bd2)
        gsem = (gsem0, gsem1, gsem2)
        wsem = (wsem0, wsem1, wsem2)

        pend_s = [None, None]
        for k in range(RPS // RCH):
            p = k % 2
            if pend_s[p] is not None:
                pend_s[p][1].wait()
            roff = s * RPS + k * RCH
            rd = pltpu.async_copy(h_hbm.at[pl.ds(roff, RCH)],
                                  bs[p].at[pl.ds(0, RCH)], gsem[p])
            rd.wait()
            pend_s[p] = (roff, pltpu.async_copy(
                bs[p].at[pl.ds(0, RCH)], h_sh.at[pl.ds(roff, RCH)], ssem))
        for p in range(2):
            if pend_s[p] is not None:
                pend_s[p][1].wait()

        @pl.when(s == 0)
        def _():
            roff = NS * RPS
            pltpu.sync_copy(h_hbm.at[pl.ds(roff, RTAIL)],
                            bs0.at[pl.ds(0, RTAIL)])
            pltpu.sync_copy(bs0.at[pl.ds(0, RTAIL)],
                            h_sh.at[pl.ds(roff, RTAIL)])

        pltpu.sync_copy(src_hbm.at[pl.ds(base, epw)], six)
        pltpu.sync_copy(dst_hbm.at[pl.ds(base, epw)], dix)
        plsc.subcore_barrier()

        pend_g = [None, None, None]
        pend_w = [None, None, None]

        def fire_gather(j, p):
            isl_s = six.at[pl.ds(j * CHG, CHG)]
            isl_d = dix.at[pl.ds(j * CHG, CHG)]
            pend_g[p] = (pltpu.async_copy(h_sh.at[isl_s], bs[p], gsem[p]),
                         pltpu.async_copy(h_sh.at[isl_d], bd[p], gsem[p]))

        def fire_writes(j, p):
            off = obase + j * CHG
            pend_w[p] = (
                pltpu.async_copy(bs[p], hs_hbm.at[pl.ds(off, CHG)], wsem[p]),
                pltpu.async_copy(bd[p], hd_hbm.at[pl.ds(off, CHG)], wsem[p]))

        fire_gather(0, 0)
        if nfull > 1:
            fire_gather(1, 1)
        for j in range(nfull):
            p = j % 3
            for dsc in pend_g[p]:
                dsc.wait()
            fire_writes(j, p)
            nj = j + 2
            if nj < nfull:
                q = nj % 3
                if pend_w[q] is not None:
                    for dsc in pend_w[q]:
                        dsc.wait()
                    pend_w[q] = None
                fire_gather(nj, q)
        for p in range(3):
            if pend_w[p] is not None:
                for dsc in pend_w[p]:
                    dsc.wait()

        if tail:
            off = nfull * CHG
            pltpu.sync_copy(src_hbm.at[pl.ds(base + off, tail)], idx_st)
            pltpu.sync_copy(dst_hbm.at[pl.ds(base + off, tail)], idx_dt)
            cps = pltpu.async_copy(h_sh.at[idx_st], bs0.at[pl.ds(0, tail)],
                                   gsem0)
            cpd = pltpu.async_copy(h_sh.at[idx_dt], bd0.at[pl.ds(0, tail)],
                                   gsem1)
            cps.wait()
            cpd.wait()
            pltpu.sync_copy(bs0.at[pl.ds(0, tail)],
                            hs_hbm.at[pl.ds(obase + off, tail)])
            pltpu.sync_copy(bd0.at[pl.ds(0, tail)],
                            hd_hbm.at[pl.ds(obase + off, tail)])

    mesh = plsc.VectorSubcoreMesh(core_axis_name="c", subcore_axis_name="s")
    f = pl.kernel(
        body,
        out_type=[jax.ShapeDtypeStruct((ne, D), jnp.float32),
                  jax.ShapeDtypeStruct((ne, D), jnp.float32)],
        mesh=mesh,
        scratch_types=[
            pltpu.VMEM((epw,), jnp.int32),
            pltpu.VMEM((epw,), jnp.int32),
            pltpu.VMEM((CHG, D), jnp.float32),
            pltpu.VMEM((CHG, D), jnp.float32),
            pltpu.VMEM((CHG, D), jnp.float32),
            pltpu.VMEM((CHG, D), jnp.float32),
            pltpu.VMEM((CHG, D), jnp.float32),
            pltpu.VMEM((CHG, D), jnp.float32),
            pltpu.VMEM((max(tail, 8),), jnp.int32),
            pltpu.VMEM((max(tail, 8),), jnp.int32),
            pltpu.VMEM_SHARED((N, D), jnp.float32),
            pltpu.SemaphoreType.DMA,
            pltpu.SemaphoreType.DMA,
            pltpu.SemaphoreType.DMA,
            pltpu.SemaphoreType.DMA,
            pltpu.SemaphoreType.DMA,
            pltpu.SemaphoreType.DMA,
            pltpu.SemaphoreType.DMA,
        ],
    )
    return f(src, dst, h)


def _sc_scatter(dst, m, e0, ne):
    epw = ne // NW
    nfull = epw // CH
    tail = epw - nfull * CH

    def body(dst_hbm, m_hbm, zrow_hbm, zcnt_hbm, one_hbm, onet_hbm,
             aggp_hbm, cntp_hbm,
             idx0, idx1, mb0, mb1, ones_v, idx_t, ones_t,
             st0, st1, cstage, agg_sh, cnt_sh,
             msem0, msem1, asem0, asem1, osem0, osem1):
        c = lax.axis_index("c")
        s = lax.axis_index("s")
        base = e0 + c * (ne // NC) + s * epw
        idxb = (idx0, idx1)
        mb = (mb0, mb1)
        st = (st0, st1)
        msem = (msem0, msem1)
        asem = (asem0, asem1)
        osem = (osem0, osem1)

        pltpu.sync_copy(zrow_hbm, st0)
        pltpu.sync_copy(zcnt_hbm, cstage)
        zp = [pltpu.async_copy(cstage, cnt_sh.at[pl.ds(s * RPS, RPS)], osem1)]
        for k in range(RPS // RCH):
            zp.append(pltpu.async_copy(
                st0, agg_sh.at[pl.ds(s * RPS + k * RCH, RCH)], osem0))

        @pl.when(s == 0)
        def _():
            pltpu.sync_copy(st0.at[pl.ds(0, RTAIL)],
                            agg_sh.at[pl.ds(NS * RPS, RTAIL)])
            pltpu.sync_copy(cstage.at[pl.ds(0, RTAIL)],
                            cnt_sh.at[pl.ds(NS * RPS, RTAIL)])

        pltpu.sync_copy(one_hbm, ones_v)
        pltpu.sync_copy(onet_hbm, ones_t)
        for dsc in zp:
            dsc.wait()
        plsc.subcore_barrier()

        pend_in = [None, None]
        pend_add = [None, None]

        def fire_in(j, p):
            off = base + j * CH
            pend_in[p] = (
                pltpu.async_copy(dst_hbm.at[pl.ds(off, CH)], idxb[p], msem[p]),
                pltpu.async_copy(m_hbm.at[pl.ds(j * CH + base - e0, CH)],
                                 mb[p], msem[p]))

        def fire_add(j, p):
            pend_add[p] = (
                pltpu.async_copy(mb[p], agg_sh.at[idxb[p]], asem[p], add=True),
                pltpu.async_copy(ones_v, cnt_sh.at[idxb[p]], asem[p],
                                 add=True))

        fire_in(0, 0)
        if nfull > 1:
            fire_in(1, 1)
        for j in range(nfull):
            p = j % 2
            for dsc in pend_in[p]:
                dsc.wait()
            if pend_add[p] is not None:
                for dsc in pend_add[p]:
                    dsc.wait()
            fire_add(j, p)
            nj = j + 2
            if nj < nfull:
                for dsc in pend_add[p]:
                    dsc.wait()
                pend_add[p] = None
                fire_in(nj, p)
        for p in range(2):
            if pend_add[p] is not None:
                for dsc in pend_add[p]:
                    dsc.wait()

        if tail:
            off = base + nfull * CH
            pltpu.sync_copy(dst_hbm.at[pl.ds(off, tail)], idx_t)
            pltpu.sync_copy(m_hbm.at[pl.ds(off - e0, tail)],
                            mb0.at[pl.ds(0, tail)])
            pltpu.sync_copy(mb0.at[pl.ds(0, tail)], agg_sh.at[idx_t],
                            add=True)
            pltpu.sync_copy(ones_t, cnt_sh.at[idx_t], add=True)

        plsc.subcore_barrier()

        pend_rd = [None, None]

        def fire_rd(k, p):
            pend_rd[p] = pltpu.async_copy(
                agg_sh.at[pl.ds(s * RPS + k * RCH, RCH)], st[p], msem[p])

        pend_wr = [None, None]
        fire_rd(0, 0)
        fire_rd(1, 1)
        for k in range(RPS // RCH):
            p = k % 2
            pend_rd[p].wait()
            if pend_wr[p] is not None:
                pend_wr[p].wait()
            pend_wr[p] = pltpu.async_copy(
                st[p], aggp_hbm.at[c, pl.ds(s * RPS + k * RCH, RCH)], osem[p])
            nk = k + 2
            if nk < RPS // RCH:
                pend_wr[p].wait()
                pend_wr[p] = None
                fire_rd(nk, p)
        for p in range(2):
            if pend_wr[p] is not None:
                pend_wr[p].wait()
        pltpu.sync_copy(cnt_sh.at[pl.ds(s * RPS, RPS)], cstage)
        pltpu.sync_copy(cstage, cntp_hbm.at[pl.ds(c * N + s * RPS, RPS)])

        @pl.when(s == 0)
        def _():
            pltpu.sync_copy(agg_sh.at[pl.ds(NS * RPS, RTAIL)],
                            st0.at[pl.ds(0, RTAIL)])
            pltpu.sync_copy(st0.at[pl.ds(0, RTAIL)],
                            aggp_hbm.at[c, pl.ds(NS * RPS, RTAIL)])
            pltpu.sync_copy(cnt_sh.at[pl.ds(NS * RPS, RTAIL)],
                            cstage.at[pl.ds(0, RTAIL)])
            pltpu.sync_copy(cstage.at[pl.ds(0, RTAIL)],
                            cntp_hbm.at[pl.ds(c * N + NS * RPS, RTAIL)])

    zrow = jnp.zeros((RCH, D), jnp.float32)
    zcnt = jnp.zeros((RPS,), jnp.float32)
    one = jnp.ones((CH,), jnp.float32)
    onet = jnp.ones((max(tail, 8),), jnp.float32)
    mesh = plsc.VectorSubcoreMesh(core_axis_name="c", subcore_axis_name="s")
    f = pl.kernel(
        body,
        out_type=[jax.ShapeDtypeStruct((NC, N, D), jnp.float32),
                  jax.ShapeDtypeStruct((NC * N,), jnp.float32)],
        mesh=mesh,
        scratch_types=[
            pltpu.VMEM((CH,), jnp.int32),
            pltpu.VMEM((CH,), jnp.int32),
            pltpu.VMEM((CH, D), jnp.float32),
            pltpu.VMEM((CH, D), jnp.float32),
            pltpu.VMEM((CH,), jnp.float32),
            pltpu.VMEM((max(tail, 8),), jnp.int32),
            pltpu.VMEM((max(tail, 8),), jnp.float32),
            pltpu.VMEM((RCH, D), jnp.float32),
            pltpu.VMEM((RCH, D), jnp.float32),
            pltpu.VMEM((RPS,), jnp.float32),
            pltpu.VMEM_SHARED((N, D), jnp.float32),
            pltpu.VMEM_SHARED((N,), jnp.float32),
            pltpu.SemaphoreType.DMA,
            pltpu.SemaphoreType.DMA,
            pltpu.SemaphoreType.DMA,
            pltpu.SemaphoreType.DMA,
            pltpu.SemaphoreType.DMA,
            pltpu.SemaphoreType.DMA,
        ],
    )
    return f(dst, m, zrow, zcnt, one, onet)


def _edge_mlp_body(hs, hd, ea, w1s, w1d, w1e, b1, w2, b2, w3, b3, out):
    bf = jnp.bfloat16
    x = (jnp.dot(hs[...].astype(bf), w1s[...].astype(bf),
                 preferred_element_type=jnp.float32)
         + jnp.dot(hd[...].astype(bf), w1d[...].astype(bf),
                   preferred_element_type=jnp.float32)
         + lax.dot_general(ea[...].astype(bf), w1e[...].astype(bf),
                           (((0,), (0,)), ((), ())),
                           preferred_element_type=jnp.float32)
         + b1[...])
    x = _gelu(x)
    x = _gelu(jnp.dot(x.astype(bf), w2[...].astype(bf),
                      preferred_element_type=jnp.float32) + b2[...])
    out[...] = jnp.dot(x.astype(bf), w3[...].astype(bf),
                       preferred_element_type=jnp.float32) + b3[...]


def _tc_edge_mlp(hs, hd, ea, W1, b1, W2, b2, W3, b3, BE, ne, e0):
    grid = (ne // BE,)
    eoff = e0 // BE
    w1s, w1d, w1e = W1[:D], W1[D:2 * D], W1[2 * D:]
    full = lambda shape: pl.BlockSpec(shape, lambda i: (0, 0))
    return pl.pallas_call(
        _edge_mlp_body,
        grid=grid,
        in_specs=[
            pl.BlockSpec((BE, D), lambda i: (i, 0)),
            pl.BlockSpec((BE, D), lambda i: (i, 0)),
            pl.BlockSpec((ED, BE), lambda i: (0, i + eoff)),
            full((D, D)), full((D, D)), full((ED, D)), full((1, D)),
            full((D, D)), full((1, D)),
            full((D, D)), full((1, D)),
        ],
        out_specs=pl.BlockSpec((BE, D), lambda i: (i, 0)),
        out_shape=jax.ShapeDtypeStruct((ne, D), jnp.float32),
    )(hs, hd, ea, w1s, w1d, w1e, b1.reshape(1, D), W2, b2.reshape(1, D),
      W3, b3.reshape(1, D))


def _node_body(h, a0, a1, a2, a3, c0, c1, c2, c3,
               u1h, u1a, ub1, u2, ub2, lw, lb, out):
    cnt = c0[...] + c1[...] + c2[...] + c3[...] + jnp.float32(1e-8)
    agg = (a0[...] + a1[...] + a2[...] + a3[...]) / cnt
    u = _gelu(jnp.dot(h[...], u1h[...], preferred_element_type=jnp.float32)
              + jnp.dot(agg, u1a[...], preferred_element_type=jnp.float32)
              + ub1[...])
    x = jnp.dot(u, u2[...], preferred_element_type=jnp.float32) + ub2[...] + h[...]
    mu = jnp.mean(x, axis=-1, keepdims=True)
    xc = x - mu
    var = jnp.mean(xc * xc, axis=-1, keepdims=True)
    out[...] = xc * lax.rsqrt(var + jnp.float32(1e-5)) * lw[...] + lb[...]


def _tc_node(h, aggs, cnts, U1, ub1, U2, ub2, ln_w, ln_b):
    BN = 2000
    grid = (N // BN,)
    u1h, u1a = U1[:D], U1[D:]
    full = lambda shape: pl.BlockSpec(shape, lambda i: (0, 0))
    row = pl.BlockSpec((BN, D), lambda i: (i, 0))
    col = pl.BlockSpec((BN, 1), lambda i: (i, 0))
    return pl.pallas_call(
        _node_body,
        grid=grid,
        in_specs=[row] + [row] * 4 + [col] * 4 + [
                  full((D, D)), full((D, D)), full((1, D)),
                  full((D, D)), full((1, D)),
                  full((1, D)), full((1, D))],
        out_specs=row,
        out_shape=jax.ShapeDtypeStruct((N, D), jnp.float32),
    )(h, *aggs, *cnts,
      u1h, u1a, ub1.reshape(1, D), U2, ub2.reshape(1, D),
      ln_w.reshape(1, D), ln_b.reshape(1, D))


def kernel(h, edge_index, edge_attr, W1, b1, W2, b2, W3, b3, U1, ub1, U2, ub2,
           ln_w, ln_b):
    src = edge_index[0]
    dst = edge_index[1]
    ea_t = edge_attr.T
    chunks = ((0, 160000, 3200), (160000, 160000, 3200))
    aggs, cnts = [], []
    for e0, ne, be in chunks:
        hs, hd = _sc_gather(src, dst, h, e0, ne)
        m = _tc_edge_mlp(hs, hd, ea_t, W1, b1, W2, b2, W3,
                         b3, be, ne, e0)
        aggp, cntp = _sc_scatter(dst, m, e0, ne)
        aggs += [aggp[0], aggp[1]]
        cnts += [cntp[:N].reshape(N, 1), cntp[N:].reshape(N, 1)]
    return _tc_node(h, aggs, cnts, U1, ub1, U2, ub2, ln_w, ln_b)
